# Initial kernel scaffold; baseline (speedup 1.0000x reference)
import jax
import jax.numpy as jnp
from jax.experimental import pallas as pl


def kernel(x, edge_index, W1, a_src1, a_dst1, b1, W2, a_src2, a_dst2, b2, actor_W, actor_b, critic_W, critic_b):
    raise NotImplementedError("write your pallas kernel here")


# speedup vs baseline: 1.0353x; 1.0353x over previous
import jax
import jax.numpy as jnp
from jax.experimental import pallas as pl

_N = 2048
_R = 256
_NEG = -3.0e38


def _prep1_kernel(xc_ref, x_ref, cw_ref, w1_ref, asrc_ref, adst_ref,
                  h_ref, ut_ref, wcol_ref, vacc_ref):
    xc = xc_ref[...]
    h = jnp.dot(xc, w1_ref[...], preferred_element_type=jnp.float32)
    h_ref[...] = h
    u = jnp.dot(h, asrc_ref[...], preferred_element_type=jnp.float32)
    wcol_ref[...] = jnp.dot(h, adst_ref[...], preferred_element_type=jnp.float32)
    ut_ref[...] = jnp.transpose(u)
    vacc_ref[...] = jnp.sum(x_ref[...] * cw_ref[...]).reshape(1, 1)


def _gat1_kernel(c0_ref, h_ref, ut_ref, wcol_ref, b_ref, out_ref):
    i = pl.program_id(0)
    c = c0_ref[...]
    rows = i * _R + jax.lax.broadcasted_iota(jnp.int32, (_R, _N), 0)
    cols = jax.lax.broadcasted_iota(jnp.int32, (_R, _N), 1)
    c = c + jnp.where(rows == cols, 1.0, 0.0)
    s = wcol_ref[...] + ut_ref[...]
    lk = jnp.where(s >= 0, s, 0.2 * s)
    masked = jnp.where(c > 0, lk, _NEG)
    m = jnp.max(masked, axis=1, keepdims=True)
    p = jnp.where(c > 0, c * jnp.exp(lk - m), 0.0)
    denom = jnp.sum(p, axis=1, keepdims=True) + 1e-16
    h = h_ref[...]
    out_ref[...] = jnp.dot(p, h, preferred_element_type=jnp.float32) / denom \
        + b_ref[...]


def _prep2_kernel(h1_ref, w2_ref, asrc_ref, adst_ref,
                  h_ref, ut_ref, wcol_ref):
    h1 = h1_ref[...]
    g = jnp.where(h1 > 0, h1, jnp.exp(h1) - 1.0)
    h = jnp.dot(g, w2_ref[...], preferred_element_type=jnp.float32)
    h_ref[...] = h
    u = jnp.dot(h, asrc_ref[...], preferred_element_type=jnp.float32)
    wcol_ref[...] = jnp.dot(h, adst_ref[...], preferred_element_type=jnp.float32)
    ut_ref[...] = jnp.transpose(u)


def _gat2_kernel(c0_ref, h_ref, ut_ref, wcol_ref, b_ref, aw_ref, lacc_ref):
    i = pl.program_id(0)
    c = c0_ref[...]
    rows = i * _R + jax.lax.broadcasted_iota(jnp.int32, (_R, _N), 0)
    cols = jax.lax.broadcasted_iota(jnp.int32, (_R, _N), 1)
    c = c + jnp.where(rows == cols, 1.0, 0.0)
    s = wcol_ref[...] + ut_ref[...]
    lk = jnp.where(s >= 0, s, 0.2 * s)
    masked = jnp.where(c > 0, lk, _NEG)
    m = jnp.max(masked, axis=1, keepdims=True)
    p = jnp.where(c > 0, c * jnp.exp(lk - m), 0.0)
    denom = jnp.sum(p, axis=1, keepdims=True) + 1e-16
    h2 = jnp.dot(p, h_ref[...], preferred_element_type=jnp.float32) / denom \
        + b_ref[...]
    part = jnp.sum(h2 * aw_ref[...]).reshape(1, 1)

    @pl.when(i == 0)
    def _():
        lacc_ref[...] = jnp.zeros((1, 1), jnp.float32)

    lacc_ref[...] += part


def kernel(x, edge_index, W1, a_src1, a_dst1, b1, W2, a_src2, a_dst2, b2,
           actor_W, actor_b, critic_W, critic_b):
    C0 = jnp.zeros((_N, _N), jnp.float32).at[edge_index[1], edge_index[0]].add(1.0)
    A = jnp.minimum(jnp.maximum(C0, C0.T), 1.0)
    deg = A.sum(axis=1)
    dis = jnp.where(deg > 0, 1.0 / jnp.sqrt(jnp.maximum(deg, 1e-12)), 0.0)
    L = jnp.eye(_N, dtype=jnp.float32) - dis[:, None] * A * dis[None, :]

    _, evecs = jnp.linalg.eigh(L)
    lap_ev = evecs[:, :2]
    x_combined = jnp.concatenate([x, lap_ev], axis=1)

    loops = jnp.arange(_N, dtype=edge_index.dtype)
    ei = jnp.concatenate([edge_index, jnp.stack([loops, loops])], axis=1)

    f32 = jnp.float32
    h1p, ut1, wc1, vacc = pl.pallas_call(
        _prep1_kernel,
        out_shape=(
            jax.ShapeDtypeStruct((_N, 8), f32),
            jax.ShapeDtypeStruct((1, _N), f32),
            jax.ShapeDtypeStruct((_N, 1), f32),
            jax.ShapeDtypeStruct((1, 1), f32),
        ),
    )(x_combined, x, critic_W.reshape(_N, 3), W1,
      a_src1.reshape(8, 1), a_dst1.reshape(8, 1))

    h1 = pl.pallas_call(
        _gat1_kernel,
        grid=(_N // _R,),
        in_specs=[
            pl.BlockSpec((_R, _N), lambda i: (i, 0)),
            pl.BlockSpec((_N, 8), lambda i: (0, 0)),
            pl.BlockSpec((1, _N), lambda i: (0, 0)),
            pl.BlockSpec((_R, 1), lambda i: (i, 0)),
            pl.BlockSpec((1, 8), lambda i: (0, 0)),
        ],
        out_specs=pl.BlockSpec((_R, 8), lambda i: (i, 0)),
        out_shape=jax.ShapeDtypeStruct((_N, 8), f32),
    )(C0, h1p, ut1, wc1, b1.reshape(1, 8))

    h2p, ut2, wc2 = pl.pallas_call(
        _prep2_kernel,
        out_shape=(
            jax.ShapeDtypeStruct((_N, 5), f32),
            jax.ShapeDtypeStruct((1, _N), f32),
            jax.ShapeDtypeStruct((_N, 1), f32),
        ),
    )(h1, W2, a_src2.reshape(5, 1), a_dst2.reshape(5, 1))

    lacc = pl.pallas_call(
        _gat2_kernel,
        grid=(_N // _R,),
        in_specs=[
            pl.BlockSpec((_R, _N), lambda i: (i, 0)),
            pl.BlockSpec((_N, 5), lambda i: (0, 0)),
            pl.BlockSpec((1, _N), lambda i: (0, 0)),
            pl.BlockSpec((_R, 1), lambda i: (i, 0)),
            pl.BlockSpec((1, 5), lambda i: (0, 0)),
            pl.BlockSpec((_R, 5), lambda i: (i, 0)),
        ],
        out_specs=pl.BlockSpec((1, 1), lambda i: (0, 0)),
        out_shape=jax.ShapeDtypeStruct((1, 1), f32),
    )(C0, h2p, ut2, wc2, b2.reshape(1, 5), actor_W.reshape(_N, 5))

    logits = lacc[0, 0] + actor_b
    value = vacc[0, 0] + critic_b
    return (logits, value, x_combined, ei)


# speedup vs baseline: 1.0357x; 1.0004x over previous
import functools

import jax
import jax.numpy as jnp
from jax import lax
from jax.experimental import pallas as pl
from jax.experimental.pallas import tpu as pltpu
from jax.experimental.pallas import tpu_sc as plsc

_N = 2048
_E = 32768
_R = 256
_NEG = -3.0e38

_NC = 2
_NS = 16
_NW = _NC * _NS
_ROWS = 32
_ROUNDS = _N // (_NW * _ROWS)
_CHUNK = 2048
_NCHUNK = _E // _CHUNK


_CH = 512
_CHW = _CH * _N
_PASSES = 2
_SHARE = _E // _NS
_ZW = _CHW // _NS


def _c0_sc_kernel(src_hbm, dst_hbm, zch_hbm, ones_hbm, out_hbm,
                  shared, src_v, dst_v, idx_v, ones_v):
    c = lax.axis_index("c")
    s = lax.axis_index("s")
    base = s * _SHARE
    pltpu.sync_copy(src_hbm.at[pl.ds(base, _SHARE)], src_v)
    pltpu.sync_copy(dst_hbm.at[pl.ds(base, _SHARE)], dst_v)
    pltpu.sync_copy(ones_hbm, ones_v)
    zoff = s * _ZW

    for p in range(_PASSES):
        row0 = (c * _PASSES + p) * _CH
        pltpu.sync_copy(zch_hbm, shared.at[pl.ds(zoff, _ZW)])
        plsc.subcore_barrier()

        def vec_body(j, carry):
            s16 = src_v[pl.ds(j * 16, 16)]
            d16 = dst_v[pl.ds(j * 16, 16)]
            rel = d16 - row0
            mask = (rel >= 0) & (rel < _CH)
            idx_v[j // 8, pl.ds((j % 8) * 16, 16)] = \
                jnp.where(mask, rel * _N + s16, _CHW)
            return carry

        for j in range(_SHARE // 16):
            vec_body(j, 0)

        def tile_turn(t, carry):
            @pl.when(s == t)
            def _():
                def dma_row(jj, c2):
                    pltpu.sync_copy(ones_v, shared.at[idx_v.at[jj]], add=True)
                    return c2
                lax.fori_loop(0, _SHARE // 128, dma_row, 0)
            plsc.subcore_barrier()
            return carry

        lax.fori_loop(0, _NS, tile_turn, 0)
        plsc.subcore_barrier()
        pltpu.sync_copy(shared.at[pl.ds(zoff, _ZW)],
                        out_hbm.at[pl.ds(row0 * _N + zoff, _ZW)])
        plsc.subcore_barrier()


def _build_c0(edge_index):
    mesh = plsc.VectorSubcoreMesh(core_axis_name="c", subcore_axis_name="s",
                                  num_cores=_NC, num_subcores=_NS)
    k = functools.partial(
        pl.kernel,
        mesh=mesh,
        out_type=jax.ShapeDtypeStruct((_N * _N,), jnp.float32),
        scratch_types=[
            pltpu.VMEM_SHARED((_CHW + 16,), jnp.float32),
            pltpu.VMEM((_SHARE,), jnp.int32),
            pltpu.VMEM((_SHARE,), jnp.int32),
            pltpu.VMEM((_SHARE // 128, 128), jnp.int32),
            pltpu.VMEM((128,), jnp.float32),
        ],
    )(_c0_sc_kernel)
    zch = jnp.zeros((_ZW,), jnp.float32)
    ones = jnp.ones((128,), jnp.float32)
    return k(edge_index[0], edge_index[1], zch, ones).reshape(_N, _N)


def _adeg_kernel(c0r_ref, c0c_ref, a_ref, deg_ref):
    a = jnp.minimum(jnp.maximum(c0r_ref[...], jnp.transpose(c0c_ref[...])), 1.0)
    a_ref[...] = a
    deg_ref[...] = jnp.sum(a, axis=1, keepdims=True)


def _lap_kernel(a_ref, disr_ref, disc_ref, l_ref):
    i = pl.program_id(0)
    rows = i * _R + lax.broadcasted_iota(jnp.int32, (_R, _N), 0)
    cols = lax.broadcasted_iota(jnp.int32, (_R, _N), 1)
    eye = jnp.where(rows == cols, 1.0, 0.0)
    t = disr_ref[...] * a_ref[...]
    t = t * disc_ref[...]
    l_ref[...] = eye - t


def _prep1_kernel(xc_ref, x_ref, cw_ref, w1_ref, asrc_ref, adst_ref,
                  h_ref, ut_ref, wcol_ref, vacc_ref):
    xc = xc_ref[...]
    h = jnp.dot(xc, w1_ref[...], preferred_element_type=jnp.float32)
    h_ref[...] = h
    u = jnp.dot(h, asrc_ref[...], preferred_element_type=jnp.float32)
    wcol_ref[...] = jnp.dot(h, adst_ref[...], preferred_element_type=jnp.float32)
    ut_ref[...] = jnp.transpose(u)
    vacc_ref[...] = jnp.sum(x_ref[...] * cw_ref[...]).reshape(1, 1)


def _gat1_kernel(c0_ref, h_ref, ut_ref, wcol_ref, b_ref, out_ref):
    i = pl.program_id(0)
    c = c0_ref[...]
    rows = i * _R + lax.broadcasted_iota(jnp.int32, (_R, _N), 0)
    cols = lax.broadcasted_iota(jnp.int32, (_R, _N), 1)
    c = c + jnp.where(rows == cols, 1.0, 0.0)
    s = wcol_ref[...] + ut_ref[...]
    lk = jnp.where(s >= 0, s, 0.2 * s)
    masked = jnp.where(c > 0, lk, _NEG)
    m = jnp.max(masked, axis=1, keepdims=True)
    p = jnp.where(c > 0, c * jnp.exp(lk - m), 0.0)
    denom = jnp.sum(p, axis=1, keepdims=True) + 1e-16
    h = h_ref[...]
    out_ref[...] = jnp.dot(p, h, preferred_element_type=jnp.float32) / denom \
        + b_ref[...]


def _prep2_kernel(h1_ref, w2_ref, asrc_ref, adst_ref,
                  h_ref, ut_ref, wcol_ref):
    h1 = h1_ref[...]
    g = jnp.where(h1 > 0, h1, jnp.exp(h1) - 1.0)
    h = jnp.dot(g, w2_ref[...], preferred_element_type=jnp.float32)
    h_ref[...] = h
    u = jnp.dot(h, asrc_ref[...], preferred_element_type=jnp.float32)
    wcol_ref[...] = jnp.dot(h, adst_ref[...], preferred_element_type=jnp.float32)
    ut_ref[...] = jnp.transpose(u)


def _gat2_kernel(c0_ref, h_ref, ut_ref, wcol_ref, b_ref, aw_ref, lacc_ref):
    i = pl.program_id(0)
    c = c0_ref[...]
    rows = i * _R + lax.broadcasted_iota(jnp.int32, (_R, _N), 0)
    cols = lax.broadcasted_iota(jnp.int32, (_R, _N), 1)
    c = c + jnp.where(rows == cols, 1.0, 0.0)
    s = wcol_ref[...] + ut_ref[...]
    lk = jnp.where(s >= 0, s, 0.2 * s)
    masked = jnp.where(c > 0, lk, _NEG)
    m = jnp.max(masked, axis=1, keepdims=True)
    p = jnp.where(c > 0, c * jnp.exp(lk - m), 0.0)
    denom = jnp.sum(p, axis=1, keepdims=True) + 1e-16
    h2 = jnp.dot(p, h_ref[...], preferred_element_type=jnp.float32) / denom \
        + b_ref[...]
    part = jnp.sum(h2 * aw_ref[...]).reshape(1, 1)

    @pl.when(i == 0)
    def _():
        lacc_ref[...] = jnp.zeros((1, 1), jnp.float32)

    lacc_ref[...] += part


def kernel(x, edge_index, W1, a_src1, a_dst1, b1, W2, a_src2, a_dst2, b2,
           actor_W, actor_b, critic_W, critic_b):
    f32 = jnp.float32

    C0 = _build_c0(edge_index)

    A, deg = pl.pallas_call(
        _adeg_kernel,
        grid=(_N // _R,),
        in_specs=[
            pl.BlockSpec((_R, _N), lambda i: (i, 0)),
            pl.BlockSpec((_N, _R), lambda i: (0, i)),
        ],
        out_specs=(
            pl.BlockSpec((_R, _N), lambda i: (i, 0)),
            pl.BlockSpec((_R, 1), lambda i: (i, 0)),
        ),
        out_shape=(
            jax.ShapeDtypeStruct((_N, _N), f32),
            jax.ShapeDtypeStruct((_N, 1), f32),
        ),
    )(C0, C0)

    dis = jnp.where(deg > 0, 1.0 / jnp.sqrt(jnp.maximum(deg, 1e-12)), 0.0)

    L = pl.pallas_call(
        _lap_kernel,
        grid=(_N // _R,),
        in_specs=[
            pl.BlockSpec((_R, _N), lambda i: (i, 0)),
            pl.BlockSpec((_R, 1), lambda i: (i, 0)),
            pl.BlockSpec((1, _N), lambda i: (0, 0)),
        ],
        out_specs=pl.BlockSpec((_R, _N), lambda i: (i, 0)),
        out_shape=jax.ShapeDtypeStruct((_N, _N), f32),
    )(A, dis, dis.reshape(1, _N))

    _, evecs = jnp.linalg.eigh(L)
    lap_ev = evecs[:, :2]
    x_combined = jnp.concatenate([x, lap_ev], axis=1)

    loops = jnp.arange(_N, dtype=edge_index.dtype)
    ei = jnp.concatenate([edge_index, jnp.stack([loops, loops])], axis=1)

    h1p, ut1, wc1, vacc = pl.pallas_call(
        _prep1_kernel,
        out_shape=(
            jax.ShapeDtypeStruct((_N, 8), f32),
            jax.ShapeDtypeStruct((1, _N), f32),
            jax.ShapeDtypeStruct((_N, 1), f32),
            jax.ShapeDtypeStruct((1, 1), f32),
        ),
    )(x_combined, x, critic_W.reshape(_N, 3), W1,
      a_src1.reshape(8, 1), a_dst1.reshape(8, 1))

    h1 = pl.pallas_call(
        _gat1_kernel,
        grid=(_N // _R,),
        in_specs=[
            pl.BlockSpec((_R, _N), lambda i: (i, 0)),
            pl.BlockSpec((_N, 8), lambda i: (0, 0)),
            pl.BlockSpec((1, _N), lambda i: (0, 0)),
            pl.BlockSpec((_R, 1), lambda i: (i, 0)),
            pl.BlockSpec((1, 8), lambda i: (0, 0)),
        ],
        out_specs=pl.BlockSpec((_R, 8), lambda i: (i, 0)),
        out_shape=jax.ShapeDtypeStruct((_N, 8), f32),
    )(C0, h1p, ut1, wc1, b1.reshape(1, 8))

    h2p, ut2, wc2 = pl.pallas_call(
        _prep2_kernel,
        out_shape=(
            jax.ShapeDtypeStruct((_N, 5), f32),
            jax.ShapeDtypeStruct((1, _N), f32),
            jax.ShapeDtypeStruct((_N, 1), f32),
        ),
    )(h1, W2, a_src2.reshape(5, 1), a_dst2.reshape(5, 1))

    lacc = pl.pallas_call(
        _gat2_kernel,
        grid=(_N // _R,),
        in_specs=[
            pl.BlockSpec((_R, _N), lambda i: (i, 0)),
            pl.BlockSpec((_N, 5), lambda i: (0, 0)),
            pl.BlockSpec((1, _N), lambda i: (0, 0)),
            pl.BlockSpec((_R, 1), lambda i: (i, 0)),
            pl.BlockSpec((1, 5), lambda i: (0, 0)),
            pl.BlockSpec((_R, 5), lambda i: (i, 0)),
        ],
        out_specs=pl.BlockSpec((1, 1), lambda i: (0, 0)),
        out_shape=jax.ShapeDtypeStruct((1, 1), f32),
    )(C0, h2p, ut2, wc2, b2.reshape(1, 5), actor_W.reshape(_N, 5))

    logits = lacc[0, 0] + actor_b
    value = vacc[0, 0] + critic_b
    return (logits, value, x_combined, ei)
